# baseline (speedup 1.0000x reference)
import jax
import jax.numpy as jnp
from jax.experimental import pallas as pl

IN_CHANNELS = 128
NY = 512
NX = 512
P = NY * NX


def kernel(voxel_features, coors):
    n = voxel_features.shape[0]
    idx = (coors[:, 2] * NX + coors[:, 3]).astype(jnp.int32)
    order = jnp.argsort(idx, stable=True)
    idx_s = idx[order]
    is_winner = jnp.concatenate([idx_s[:-1] != idx_s[1:], jnp.ones((1,), bool)])
    dest = jnp.where(is_winner, idx_s, P)
    canvas_t = jnp.zeros((P + 8, IN_CHANNELS), jnp.float32)
    canvas_t = canvas_t.at[dest, :].set(voxel_features[order])
    canvas = canvas_t[:P].T.reshape(1, IN_CHANNELS, NY, NX)
    return canvas


# speedup vs baseline: 11.7396x; 11.7396x over previous
import functools

import jax
import jax.numpy as jnp
from jax import lax
from jax.experimental import pallas as pl
from jax.experimental.pallas import tpu as pltpu
from jax.experimental.pallas import tpu_sc as plsc

C = 128
NY = 512
NX = 512
P = NY * NX
N = 30000
L = 16
NW = 32
PPW = P // NW
NPAD = ((N + L - 1) // L) * L
NCHUNK = NPAD // L
ROWS = 64
LISTCAP = PPW + ROWS
DUMP = P


def _sc_scatter_kernel(idx_hbm, vf_hbm, scr_hbm, valid_hbm,
                       idx_v, map_v, glist_v, dlist_v, g64_v, d64_v,
                       valid_v, rows_v, sem_g, sem_s):
    cid = lax.axis_index("c")
    sid = lax.axis_index("s")
    wid = sid * 2 + cid
    base = wid * PPW

    pltpu.sync_copy(idx_hbm, idx_v)

    lane = lax.iota(jnp.int32, L)
    neg1 = jnp.full((L,), -1, jnp.int32)

    def init_map(k, _):
        map_v[pl.ds(k * L, L)] = neg1
        return 0
    lax.fori_loop(0, PPW // L, init_map, 0)

    def init_lists(k, _):
        glist_v[pl.ds(k * L, L)] = lane
        dlist_v[pl.ds(k * L, L)] = DUMP + lane + (k % 4) * L
        return 0
    lax.fori_loop(0, LISTCAP // L, init_lists, 0)

    def phase1(c, _):
        p = idx_v[pl.ds(c * L, L)]
        local = p - base
        inr = (local >= 0) & (local < PPW)
        _, lastm = plsc.scan_count(local, mask=inr)
        win = lastm & inr
        loc_w = jnp.where(win, local, 0)
        jvec = c * L + lane
        plsc.store_scatter(map_v, [loc_w], jvec, mask=win)
        return 0
    lax.fori_loop(0, NCHUNK, phase1, 0)

    def phase2(k, cursor):
        w = map_v[pl.ds(k * L, L)]
        vm = w >= 0
        valid_v[pl.ds(k * L, L)] = jnp.where(vm, 1, 0).astype(jnp.int32)
        plsc.store_compressed(glist_v.at[pl.ds(cursor, L)], w, mask=vm)
        pvec = base + k * L + lane
        plsc.store_compressed(dlist_v.at[pl.ds(cursor, L)], pvec, mask=vm)
        return cursor + jnp.sum(vm.astype(jnp.int32))
    cursor = lax.fori_loop(0, PPW // L, phase2, jnp.int32(0))

    pltpu.sync_copy(valid_v, valid_hbm.at[pl.ds(base, PPW)])

    nburst = (cursor + (ROWS - 1)) // ROWS

    def phase3(b, _):
        for t in range(ROWS // L):
            g64_v[pl.ds(t * L, L)] = glist_v[pl.ds(b * ROWS + t * L, L)]
            d64_v[pl.ds(t * L, L)] = dlist_v[pl.ds(b * ROWS + t * L, L)]
        pltpu.async_copy(vf_hbm.at[g64_v], rows_v, sem_g).wait()
        pltpu.async_copy(rows_v, scr_hbm.at[d64_v], sem_s).wait()
        return 0
    lax.fori_loop(0, nburst, phase3, 0)


def _tc_transpose_body(scr_ref, val_ref, out_ref):
    blk = scr_ref[...]
    t = blk.T
    v = val_ref[0, 0, :]
    out_ref[...] = jnp.where((v != 0)[None, :], t, jnp.float32(0.0))


@jax.jit
def kernel(voxel_features, coors):
    idx = (coors[:, 2] * NX + coors[:, 3]).astype(jnp.int32)

    mesh = plsc.VectorSubcoreMesh(core_axis_name="c", subcore_axis_name="s")
    sc = functools.partial(
        pl.kernel,
        mesh=mesh,
        compiler_params=pltpu.CompilerParams(needs_layout_passes=False),
        out_type=(
            jax.ShapeDtypeStruct((P + ROWS, C), jnp.float32),
            jax.ShapeDtypeStruct((P,), jnp.int32),
        ),
        scratch_types=[
            pltpu.VMEM((NPAD,), jnp.int32),
            pltpu.VMEM((PPW,), jnp.int32),
            pltpu.VMEM((LISTCAP,), jnp.int32),
            pltpu.VMEM((LISTCAP,), jnp.int32),
            pltpu.VMEM((ROWS,), jnp.int32),
            pltpu.VMEM((ROWS,), jnp.int32),
            pltpu.VMEM((PPW,), jnp.int32),
            pltpu.VMEM((ROWS, C), jnp.float32),
            pltpu.SemaphoreType.DMA,
            pltpu.SemaphoreType.DMA,
        ],
    )(_sc_scatter_kernel)
    scr, valid = sc(idx, voxel_features)

    valid3 = valid.reshape(NW * 2, 1, PPW // 2)

    canvas2d = pl.pallas_call(
        _tc_transpose_body,
        grid=(64,),
        in_specs=[
            pl.BlockSpec((4096, C), lambda i: (i, 0)),
            pl.BlockSpec((1, 1, 4096), lambda i: (i, 0, 0)),
        ],
        out_specs=pl.BlockSpec((C, 4096), lambda i: (0, i)),
        out_shape=jax.ShapeDtypeStruct((C, P), jnp.float32),
    )(scr, valid3)

    return canvas2d.reshape(1, C, NY, NX)
